# Initial kernel scaffold; baseline (speedup 1.0000x reference)
import jax
import jax.numpy as jnp
from jax.experimental import pallas as pl


def kernel(H, lengths):
    raise NotImplementedError("write your pallas kernel here")


# speedup vs baseline: 1.4491x; 1.4491x over previous
import jax
import jax.numpy as jnp
from jax.experimental import pallas as pl
from jax.experimental.pallas import tpu as pltpu

_Q = 0.15
_TB = 512


def _scores_body(h_ref, s_ref):
    x = h_ref[0]
    s_ref[0] = jnp.sum(x * x, axis=1)[None, :]


def _select_weights(scores, length, kk, t_total):
    iota = jax.lax.broadcasted_iota(jnp.int32, scores.shape, 1)
    keys = jax.lax.bitcast_convert_type(scores, jnp.int32)
    keys = jnp.where(iota < length, keys, jnp.int32(-1))

    def bs_body(_, lohi):
        lo, hi = lohi
        mid = lo + jax.lax.shift_right_logical(hi - lo, 1)
        cnt_gt = jnp.sum((keys > mid).astype(jnp.int32))
        take_hi = cnt_gt >= kk
        return (jnp.where(take_hi, mid + 1, lo), jnp.where(take_hi, hi, mid))

    lo, _ = jax.lax.fori_loop(
        0, 32, bs_body, (jnp.int32(-1), jnp.int32(2**31 - 1))
    )
    v = lo
    cnt_gt_v = jnp.sum((keys > v).astype(jnp.int32))
    needed = kk - cnt_gt_v
    eq = keys == v

    def idx_body(_, lohi):
        lo2, hi2 = lohi
        mid = (lo2 + hi2) >> 1
        c = jnp.sum((eq & (iota <= mid)).astype(jnp.int32))
        ok = c >= needed
        return (jnp.where(ok, lo2, mid + 1), jnp.where(ok, mid, hi2))

    m, _ = jax.lax.fori_loop(
        0, 13, idx_body, (jnp.int32(0), jnp.int32(t_total - 1))
    )
    sel = (keys > v) | (eq & (iota <= m))
    return sel.astype(jnp.float32) / kk.astype(jnp.float32)


def _pool_body(len_ref, kv_ref, s_ref, h_ref, o_ref, w_ref):
    b = pl.program_id(0)
    t = pl.program_id(1)
    t_total = w_ref.shape[1]

    @pl.when(t == 0)
    def _():
        w_ref[...] = _select_weights(s_ref[0], len_ref[b], kv_ref[b], t_total)

    x = h_ref[0]
    w_chunk = w_ref[:, pl.ds(t * _TB, _TB)]
    partial = jax.lax.dot_general(
        w_chunk,
        x,
        (((1,), (0,)), ((), ())),
        precision=jax.lax.Precision.HIGHEST,
        preferred_element_type=jnp.float32,
    )

    @pl.when(t == 0)
    def _():
        o_ref[0] = partial

    @pl.when(t != 0)
    def _():
        o_ref[0] += partial


def kernel(H, lengths):
    B, T, D = H.shape
    lengths = lengths.astype(jnp.int32)
    kv = jnp.maximum(
        jnp.ceil(lengths.astype(jnp.float32) * _Q).astype(jnp.int32), 1
    )

    scores = pl.pallas_call(
        _scores_body,
        grid=(B, T // _TB),
        in_specs=[
            pl.BlockSpec((1, _TB, D), lambda b, t: (b, t, 0)),
        ],
        out_specs=pl.BlockSpec((1, 1, _TB), lambda b, t: (b, 0, t)),
        out_shape=jax.ShapeDtypeStruct((B, 1, T), jnp.float32),
    )(H)

    pooled = pl.pallas_call(
        _pool_body,
        grid=(B, T // _TB),
        in_specs=[
            pl.BlockSpec(memory_space=pltpu.SMEM),
            pl.BlockSpec(memory_space=pltpu.SMEM),
            pl.BlockSpec((1, 1, T), lambda b, t: (b, 0, 0)),
            pl.BlockSpec((1, _TB, D), lambda b, t: (b, t, 0)),
        ],
        out_specs=pl.BlockSpec((1, 1, D), lambda b, t: (b, 0, 0)),
        out_shape=jax.ShapeDtypeStruct((B, 1, D), jnp.float32),
        scratch_shapes=[pltpu.VMEM((1, T), jnp.float32)],
    )(lengths, kv, scores, H)

    return pooled.reshape(B, D)
